# baseline (speedup 1.0000x reference)
import functools

import jax
import jax.numpy as jnp
from jax import lax
from jax.experimental import pallas as pl
from jax.experimental.pallas import tpu as pltpu
from jax.experimental.pallas import tpu_sc as plsc

NC = 2
NS = 16
NW = NC * NS
N_PAD = 10240
ROWS_PER_TILE = N_PAD // NS

_sc_mesh = plsc.VectorSubcoreMesh(
    core_axis_name="c", subcore_axis_name="s", num_cores=NC, num_subcores=NS
)
_sc_params = pltpu.CompilerParams(use_tc_tiling_on_sc=False)


def _wid():
    return lax.axis_index("s") * NC + lax.axis_index("c")


K_FLIGHT = 16


def _deg_body(n_chunks, dst_hbm, ones_hbm, zeros_hbm, out_hbm,
              didx_v, ones_v, deg_sh, sem):
    cid = lax.axis_index("c")
    sid = lax.axis_index("s")
    wid = _wid()
    row0 = sid * ROWS_PER_TILE
    pltpu.sync_copy(zeros_hbm, deg_sh.at[pl.ds(row0, ROWS_PER_TILE)])
    pltpu.sync_copy(ones_hbm, ones_v)
    pltpu.sync_copy(dst_hbm.at[wid], didx_v)
    plsc.subcore_barrier()

    n_sup = n_chunks // K_FLIGHT

    def body(s, carry):
        for b in range(K_FLIGHT):
            j = s * K_FLIGHT + b
            pltpu.async_copy(ones_v, deg_sh.at[didx_v.at[j]], sem, add=True)
        for b in range(K_FLIGHT):
            j = s * K_FLIGHT + b
            pltpu.make_async_copy(ones_v, deg_sh.at[didx_v.at[j]], sem).wait()
        return carry

    lax.fori_loop(0, n_sup, body, 0)
    plsc.subcore_barrier()
    pltpu.sync_copy(deg_sh.at[pl.ds(row0, ROWS_PER_TILE)],
                    out_hbm.at[cid, pl.ds(row0, ROWS_PER_TILE)])


def _edge_body(n_chunks, table_hbm, sidx_hbm, didx_hbm, zeros_hbm, out_hbm,
               sidx_v, didx_v, rows_v, acc_sh, gsem, ssem):
    cid = lax.axis_index("c")
    sid = lax.axis_index("s")
    wid = _wid()
    row0 = sid * ROWS_PER_TILE
    pltpu.sync_copy(zeros_hbm, acc_sh.at[pl.ds(row0, ROWS_PER_TILE)])
    pltpu.sync_copy(sidx_hbm.at[wid], sidx_v)
    pltpu.sync_copy(didx_hbm.at[wid], didx_v)
    plsc.subcore_barrier()

    n_sup = n_chunks // K_FLIGHT

    for b in range(K_FLIGHT):
        pltpu.async_copy(table_hbm.at[sidx_v.at[b]], rows_v.at[0, b], gsem)

    def body(s, carry):
        bank = lax.rem(s, 2)
        nxt = 1 - bank

        @pl.when(s + 1 < n_sup)
        def _():
            for b in range(K_FLIGHT):
                j = (s + 1) * K_FLIGHT + b
                pltpu.async_copy(table_hbm.at[sidx_v.at[j]],
                                 rows_v.at[nxt, b], gsem)

        for b in range(K_FLIGHT):
            j = s * K_FLIGHT + b
            pltpu.make_async_copy(table_hbm.at[sidx_v.at[j]],
                                  rows_v.at[bank, b], gsem).wait()
        for b in range(K_FLIGHT):
            j = s * K_FLIGHT + b
            pltpu.async_copy(rows_v.at[bank, b],
                             acc_sh.at[didx_v.at[j]], ssem, add=True)
        for b in range(K_FLIGHT):
            j = s * K_FLIGHT + b
            pltpu.make_async_copy(rows_v.at[bank, b],
                                  acc_sh.at[didx_v.at[j]], ssem).wait()
        return carry

    lax.fori_loop(0, n_sup, body, 0)
    plsc.subcore_barrier()
    pltpu.sync_copy(acc_sh.at[pl.ds(row0, ROWS_PER_TILE)],
                    out_hbm.at[cid, pl.ds(row0, ROWS_PER_TILE)])


def _sc_degree(dst_idx):
    n_chunks, chunk = dst_idx.shape[1], dst_idx.shape[2]
    k = pl.kernel(
        functools.partial(_deg_body, n_chunks),
        out_type=jax.ShapeDtypeStruct((NC, N_PAD, 16), jnp.float32),
        mesh=_sc_mesh,
        scratch_types=[
            pltpu.VMEM((n_chunks, chunk), jnp.int32),
            pltpu.VMEM((chunk, 16), jnp.float32),
            pltpu.VMEM_SHARED((N_PAD, 16), jnp.float32),
            pltpu.SemaphoreType.DMA,
        ],
        compiler_params=_sc_params,
        name="gcn_degree_sc",
    )
    ones = jnp.ones((chunk, 16), jnp.float32)
    zeros = jnp.zeros((ROWS_PER_TILE, 16), jnp.float32)
    return k(dst_idx, ones, zeros)


def _sc_edge_pass(table, src_idx, dst_idx, name):
    n_chunks, chunk = src_idx.shape[1], src_idx.shape[2]
    k = pl.kernel(
        functools.partial(_edge_body, n_chunks),
        out_type=jax.ShapeDtypeStruct((NC, N_PAD, 16), jnp.float32),
        mesh=_sc_mesh,
        scratch_types=[
            pltpu.VMEM((n_chunks, chunk), jnp.int32),
            pltpu.VMEM((n_chunks, chunk), jnp.int32),
            pltpu.VMEM((2, K_FLIGHT, chunk, 16), jnp.float32),
            pltpu.VMEM_SHARED((N_PAD, 16), jnp.float32),
            pltpu.SemaphoreType.DMA,
            pltpu.SemaphoreType.DMA,
        ],
        compiler_params=_sc_params,
        name=name,
    )
    zeros = jnp.zeros((ROWS_PER_TILE * 16 // 128, 128),
                      jnp.float32).reshape(ROWS_PER_TILE, 16)
    return k(table, src_idx, dst_idx, zeros)


_BLK = 2048
_PBLK = _BLK // 8
_GRID = 5
PK_ROWS = N_PAD // 8
N_REAL_PK = 1250


def _proj1_body(x_ref, w_ref, degp_ref, p1s_ref, dinv_ref):
    dinv = lax.rsqrt(degp_ref[0] + degp_ref[1] + 1.0)
    w = w_ref[...]
    for u in range(8):
        pu = jnp.dot(x_ref[:, u, :], w, preferred_element_type=jnp.float32)
        p1s_ref[:, 16 * u:16 * (u + 1)] = dinv[:, 16 * u:16 * (u + 1)] * pu
    dinv_ref[...] = dinv


def _tc_proj1(x3, W1, degp_pk):
    return pl.pallas_call(
        _proj1_body,
        grid=(_GRID,),
        in_specs=[
            pl.BlockSpec((_PBLK, 8, 128), lambda j: (j, 0, 0)),
            pl.BlockSpec((128, 16), lambda j: (0, 0)),
            pl.BlockSpec((NC, _PBLK, 128), lambda j: (0, j, 0)),
        ],
        out_specs=[
            pl.BlockSpec((_PBLK, 128), lambda j: (j, 0)),
            pl.BlockSpec((_PBLK, 128), lambda j: (j, 0)),
        ],
        out_shape=[
            jax.ShapeDtypeStruct((PK_ROWS, 128), jnp.float32),
            jax.ShapeDtypeStruct((PK_ROWS, 128), jnp.float32),
        ],
    )(x3, W1, degp_pk)


def _hidden_body(aggp_ref, p1s_ref, dinv_ref, b1_ref, h1s_ref):
    dinv = dinv_ref[...]
    s1 = dinv * (aggp_ref[0] + aggp_ref[1] + p1s_ref[...]) + b1_ref[...]
    h1s_ref[...] = dinv * jnp.maximum(s1, 0.0)


def _tc_hidden(aggp_pk, p1s_pk, dinv_pk, b1t):
    return pl.pallas_call(
        _hidden_body,
        grid=(_GRID,),
        in_specs=[
            pl.BlockSpec((NC, _PBLK, 128), lambda j: (0, j, 0)),
            pl.BlockSpec((_PBLK, 128), lambda j: (j, 0)),
            pl.BlockSpec((_PBLK, 128), lambda j: (j, 0)),
            pl.BlockSpec((1, 128), lambda j: (0, 0)),
        ],
        out_specs=pl.BlockSpec((_PBLK, 128), lambda j: (j, 0)),
        out_shape=jax.ShapeDtypeStruct((PK_ROWS, 128), jnp.float32),
    )(aggp_pk, p1s_pk, dinv_pk, b1t)


def _pool_body(aggp_ref, h1s_ref, dinv_ref, batch_ref, w2_ref, b2_ref,
               out_ref, sum_acc, cnt_acc):
    j = pl.program_id(0)

    @pl.when(j == 0)
    def _():
        sum_acc[...] = jnp.zeros_like(sum_acc)
        cnt_acc[...] = jnp.zeros_like(cnt_acc)

    dinv = dinv_ref[...]
    s2 = dinv * (aggp_ref[0] + aggp_ref[1] + h1s_ref[...])
    row = j * _PBLK + lax.broadcasted_iota(jnp.int32, (_PBLK, 1), 0)
    s2 = jnp.where(row < N_REAL_PK, s2, 0.0)
    gi = lax.broadcasted_iota(jnp.int32, (_PBLK, 64), 1)
    ones = jnp.ones((_PBLK, 1), jnp.float32)
    for u in range(8):
        mask = (batch_ref[:, 16 * u:16 * u + 1] == gi).astype(jnp.float32)
        sum_acc[...] += lax.dot_general(
            mask, s2[:, 16 * u:16 * (u + 1)], (((0,), (0,)), ((), ())),
            preferred_element_type=jnp.float32)
        cnt_acc[...] += lax.dot_general(
            mask, ones, (((0,), (0,)), ((), ())),
            preferred_element_type=jnp.float32)

    @pl.when(j == pl.num_programs(0) - 1)
    def _():
        pooled = sum_acc[...] / jnp.maximum(cnt_acc[...], 1.0)
        logits = jnp.dot(pooled, w2_ref[...],
                         preferred_element_type=jnp.float32) + b2_ref[...]
        m = jnp.max(logits, axis=1, keepdims=True)
        lse = m + jnp.log(jnp.sum(jnp.exp(logits - m), axis=1, keepdims=True))
        out_ref[...] = logits - lse


def _tc_pool(aggp_pk, h1s_pk, dinv_pk, batch_pk, W2, b2):
    return pl.pallas_call(
        _pool_body,
        grid=(_GRID,),
        in_specs=[
            pl.BlockSpec((NC, _PBLK, 128), lambda j: (0, j, 0)),
            pl.BlockSpec((_PBLK, 128), lambda j: (j, 0)),
            pl.BlockSpec((_PBLK, 128), lambda j: (j, 0)),
            pl.BlockSpec((_PBLK, 128), lambda j: (j, 0)),
            pl.BlockSpec((16, 2), lambda j: (0, 0)),
            pl.BlockSpec((1, 2), lambda j: (0, 0)),
        ],
        out_specs=pl.BlockSpec((64, 2), lambda j: (0, 0)),
        out_shape=jax.ShapeDtypeStruct((64, 2), jnp.float32),
        scratch_shapes=[
            pltpu.VMEM((64, 16), jnp.float32),
            pltpu.VMEM((64, 1), jnp.float32),
        ],
    )(aggp_pk, h1s_pk, dinv_pk, batch_pk, W2, b2)


def kernel(x, edge_index, batch, W1, b1, W2, b2):
    e = edge_index.shape[1]

    per_tile = e // NW
    chunk = 125
    n_chunks = per_tile // chunk
    ei2 = lax.optimization_barrier(edge_index)
    src = ei2[0].reshape(NW, n_chunks, chunk)
    dst = edge_index[1].reshape(NW, n_chunks, chunk)

    bpad = jnp.pad(batch, (0, N_PAD - batch.shape[0]), constant_values=64)
    batch_pk = jnp.broadcast_to(
        bpad.reshape(-1, 8, 1), (PK_ROWS, 8, 16)).reshape(-1, 128)
    b1t = jnp.tile(b1, 8).reshape(1, 128)

    degp = _sc_degree(dst)
    x3 = x.reshape(-1, 8, 128)
    p1s_pk, dinv_pk = _tc_proj1(x3, W1, degp.reshape(NC, PK_ROWS, 128))
    agg1 = _sc_edge_pass(p1s_pk.reshape(N_PAD, 16), src, dst, "gcn_edges1_sc")
    h1s_pk = _tc_hidden(agg1.reshape(NC, PK_ROWS, 128), p1s_pk, dinv_pk, b1t)
    agg2 = _sc_edge_pass(h1s_pk.reshape(N_PAD, 16), src, dst, "gcn_edges2_sc")
    return _tc_pool(agg2.reshape(NC, PK_ROWS, 128), h1s_pk, dinv_pk,
                    batch_pk, W2, b2.reshape(1, 2))


# speedup vs baseline: 1.0228x; 1.0228x over previous
import functools

import jax
import jax.numpy as jnp
from jax import lax
from jax.experimental import pallas as pl
from jax.experimental.pallas import tpu as pltpu
from jax.experimental.pallas import tpu_sc as plsc

NC = 2
NS = 16
NW = NC * NS
N_PAD = 10240
ROWS_PER_TILE = N_PAD // NS

_sc_mesh = plsc.VectorSubcoreMesh(
    core_axis_name="c", subcore_axis_name="s", num_cores=NC, num_subcores=NS
)
_sc_params = pltpu.CompilerParams(use_tc_tiling_on_sc=False)


def _wid():
    return lax.axis_index("s") * NC + lax.axis_index("c")


K_FLIGHT = 16


def _deg_body(n_chunks, dst_hbm, ones_hbm, zeros_hbm, out_hbm,
              didx_v, ones_v, deg_sh, sem):
    cid = lax.axis_index("c")
    sid = lax.axis_index("s")
    wid = _wid()
    row0 = sid * ROWS_PER_TILE
    pltpu.sync_copy(zeros_hbm, deg_sh.at[pl.ds(row0, ROWS_PER_TILE)])
    pltpu.sync_copy(ones_hbm, ones_v)
    pltpu.sync_copy(dst_hbm.at[wid], didx_v)
    plsc.subcore_barrier()

    n_sup = n_chunks // K_FLIGHT

    def body(s, carry):
        for b in range(K_FLIGHT):
            j = s * K_FLIGHT + b
            pltpu.async_copy(ones_v, deg_sh.at[didx_v.at[j]], sem, add=True)
        for b in range(K_FLIGHT):
            j = s * K_FLIGHT + b
            pltpu.make_async_copy(ones_v, deg_sh.at[didx_v.at[j]], sem).wait()
        return carry

    lax.fori_loop(0, n_sup, body, 0)
    plsc.subcore_barrier()
    pltpu.sync_copy(deg_sh.at[pl.ds(row0, ROWS_PER_TILE)],
                    out_hbm.at[cid, pl.ds(row0, ROWS_PER_TILE)])


def _edge_body(n_chunks, table_hbm, sidx_hbm, didx_hbm, zeros_hbm, out_hbm,
               sidx_v, didx_v, rows_v, acc_sh, gsem, ssem):
    cid = lax.axis_index("c")
    sid = lax.axis_index("s")
    wid = _wid()
    row0 = sid * ROWS_PER_TILE
    pltpu.sync_copy(zeros_hbm, acc_sh.at[pl.ds(row0, ROWS_PER_TILE)])
    pltpu.sync_copy(sidx_hbm.at[wid], sidx_v)
    pltpu.sync_copy(didx_hbm.at[wid], didx_v)
    plsc.subcore_barrier()

    n_sup = n_chunks // K_FLIGHT

    for b in range(K_FLIGHT):
        pltpu.async_copy(table_hbm.at[sidx_v.at[b]], rows_v.at[0, b], gsem)

    def body(s, carry):
        bank = lax.rem(s, 2)
        nxt = 1 - bank

        @pl.when(s + 1 < n_sup)
        def _():
            for b in range(K_FLIGHT):
                j = (s + 1) * K_FLIGHT + b
                pltpu.async_copy(table_hbm.at[sidx_v.at[j]],
                                 rows_v.at[nxt, b], gsem)

        for b in range(K_FLIGHT):
            j = s * K_FLIGHT + b
            pltpu.make_async_copy(table_hbm.at[sidx_v.at[j]],
                                  rows_v.at[bank, b], gsem).wait()
        for b in range(K_FLIGHT):
            j = s * K_FLIGHT + b
            pltpu.async_copy(rows_v.at[bank, b],
                             acc_sh.at[didx_v.at[j]], ssem, add=True)
        for b in range(K_FLIGHT):
            j = s * K_FLIGHT + b
            pltpu.make_async_copy(rows_v.at[bank, b],
                                  acc_sh.at[didx_v.at[j]], ssem).wait()
        return carry

    lax.fori_loop(0, n_sup, body, 0)
    plsc.subcore_barrier()
    pltpu.sync_copy(acc_sh.at[pl.ds(row0, ROWS_PER_TILE)],
                    out_hbm.at[cid, pl.ds(row0, ROWS_PER_TILE)])


def _sc_degree(dst_idx):
    n_chunks, chunk = dst_idx.shape[1], dst_idx.shape[2]
    k = pl.kernel(
        functools.partial(_deg_body, n_chunks),
        out_type=jax.ShapeDtypeStruct((NC, N_PAD, 16), jnp.float32),
        mesh=_sc_mesh,
        scratch_types=[
            pltpu.VMEM((n_chunks, chunk), jnp.int32),
            pltpu.VMEM((chunk, 16), jnp.float32),
            pltpu.VMEM_SHARED((N_PAD, 16), jnp.float32),
            pltpu.SemaphoreType.DMA,
        ],
        compiler_params=_sc_params,
        name="gcn_degree_sc",
    )
    ones = jnp.ones((chunk, 16), jnp.float32)
    zeros = jnp.zeros((ROWS_PER_TILE, 16), jnp.float32)
    return k(dst_idx, ones, zeros)


def _sc_edge_pass(table, src_idx, dst_idx, name):
    n_chunks, chunk = src_idx.shape[1], src_idx.shape[2]
    k = pl.kernel(
        functools.partial(_edge_body, n_chunks),
        out_type=jax.ShapeDtypeStruct((NC, N_PAD, 16), jnp.float32),
        mesh=_sc_mesh,
        scratch_types=[
            pltpu.VMEM((n_chunks, chunk), jnp.int32),
            pltpu.VMEM((n_chunks, chunk), jnp.int32),
            pltpu.VMEM((2, K_FLIGHT, chunk, 16), jnp.float32),
            pltpu.VMEM_SHARED((N_PAD, 16), jnp.float32),
            pltpu.SemaphoreType.DMA,
            pltpu.SemaphoreType.DMA,
        ],
        compiler_params=_sc_params,
        name=name,
    )
    zeros = jnp.zeros((ROWS_PER_TILE * 16 // 128, 128),
                      jnp.float32).reshape(ROWS_PER_TILE, 16)
    return k(table, src_idx, dst_idx, zeros)


_BLK = 2048
_PBLK = _BLK // 8
_GRID = 5
PK_ROWS = N_PAD // 8
N_REAL_PK = 1250


def _proj1_body(x_ref, w_ref, degp_ref, p1s_ref, dinv_ref):
    dinv = lax.rsqrt(degp_ref[0] + degp_ref[1] + 1.0)
    w = w_ref[...]
    for u in range(8):
        pu = jnp.dot(x_ref[:, u, :], w, preferred_element_type=jnp.float32)
        p1s_ref[:, 16 * u:16 * (u + 1)] = dinv[:, 16 * u:16 * (u + 1)] * pu
    dinv_ref[...] = dinv


def _tc_proj1(x3, W1, degp_pk):
    return pl.pallas_call(
        _proj1_body,
        grid=(_GRID,),
        in_specs=[
            pl.BlockSpec((_PBLK, 8, 128), lambda j: (j, 0, 0)),
            pl.BlockSpec((128, 16), lambda j: (0, 0)),
            pl.BlockSpec((NC, _PBLK, 128), lambda j: (0, j, 0)),
        ],
        out_specs=[
            pl.BlockSpec((_PBLK, 128), lambda j: (j, 0)),
            pl.BlockSpec((_PBLK, 128), lambda j: (j, 0)),
        ],
        out_shape=[
            jax.ShapeDtypeStruct((PK_ROWS, 128), jnp.float32),
            jax.ShapeDtypeStruct((PK_ROWS, 128), jnp.float32),
        ],
    )(x3, W1, degp_pk)


def _hidden_body(aggp_ref, p1s_ref, dinv_ref, b1_ref, h1s_ref):
    dinv = dinv_ref[...]
    s1 = dinv * (aggp_ref[0] + aggp_ref[1] + p1s_ref[...]) + b1_ref[...]
    h1s_ref[...] = dinv * jnp.maximum(s1, 0.0)


def _tc_hidden(aggp_pk, p1s_pk, dinv_pk, b1t):
    return pl.pallas_call(
        _hidden_body,
        grid=(_GRID,),
        in_specs=[
            pl.BlockSpec((NC, _PBLK, 128), lambda j: (0, j, 0)),
            pl.BlockSpec((_PBLK, 128), lambda j: (j, 0)),
            pl.BlockSpec((_PBLK, 128), lambda j: (j, 0)),
            pl.BlockSpec((1, 128), lambda j: (0, 0)),
        ],
        out_specs=pl.BlockSpec((_PBLK, 128), lambda j: (j, 0)),
        out_shape=jax.ShapeDtypeStruct((PK_ROWS, 128), jnp.float32),
    )(aggp_pk, p1s_pk, dinv_pk, b1t)


def _pool_body(aggp_ref, h1s_ref, dinv_ref, batch_ref, w2_ref, b2_ref,
               out_ref, sum_acc, cnt_acc):
    j = pl.program_id(0)

    @pl.when(j == 0)
    def _():
        sum_acc[...] = jnp.zeros_like(sum_acc)
        cnt_acc[...] = jnp.zeros_like(cnt_acc)

    dinv = dinv_ref[...]
    s2 = dinv * (aggp_ref[0] + aggp_ref[1] + h1s_ref[...])
    row = j * _PBLK + lax.broadcasted_iota(jnp.int32, (_PBLK, 1), 0)
    s2 = jnp.where(row < N_REAL_PK, s2, 0.0)
    gi = lax.broadcasted_iota(jnp.int32, (_PBLK, 64), 1)
    ones = jnp.ones((_PBLK, 1), jnp.float32)
    for u in range(8):
        mask = (batch_ref[:, 16 * u:16 * u + 1] == gi).astype(jnp.float32)
        sum_acc[...] += lax.dot_general(
            mask, s2[:, 16 * u:16 * (u + 1)], (((0,), (0,)), ((), ())),
            preferred_element_type=jnp.float32)
        cnt_acc[...] += lax.dot_general(
            mask, ones, (((0,), (0,)), ((), ())),
            preferred_element_type=jnp.float32)

    @pl.when(j == pl.num_programs(0) - 1)
    def _():
        pooled = sum_acc[...] / jnp.maximum(cnt_acc[...], 1.0)
        logits = jnp.dot(pooled, w2_ref[...],
                         preferred_element_type=jnp.float32) + b2_ref[...]
        m = jnp.max(logits, axis=1, keepdims=True)
        lse = m + jnp.log(jnp.sum(jnp.exp(logits - m), axis=1, keepdims=True))
        out_ref[...] = logits - lse


def _tc_pool(aggp_pk, h1s_pk, dinv_pk, batch_pk, W2, b2):
    return pl.pallas_call(
        _pool_body,
        grid=(_GRID,),
        in_specs=[
            pl.BlockSpec((NC, _PBLK, 128), lambda j: (0, j, 0)),
            pl.BlockSpec((_PBLK, 128), lambda j: (j, 0)),
            pl.BlockSpec((_PBLK, 128), lambda j: (j, 0)),
            pl.BlockSpec((_PBLK, 128), lambda j: (j, 0)),
            pl.BlockSpec((16, 2), lambda j: (0, 0)),
            pl.BlockSpec((1, 2), lambda j: (0, 0)),
        ],
        out_specs=pl.BlockSpec((64, 2), lambda j: (0, 0)),
        out_shape=jax.ShapeDtypeStruct((64, 2), jnp.float32),
        scratch_shapes=[
            pltpu.VMEM((64, 16), jnp.float32),
            pltpu.VMEM((64, 1), jnp.float32),
        ],
    )(aggp_pk, h1s_pk, dinv_pk, batch_pk, W2, b2)


def kernel(x, edge_index, batch, W1, b1, W2, b2):
    e = edge_index.shape[1]

    per_tile = e // NW
    chunk = 125
    n_chunks = per_tile // chunk
    src = edge_index[0].reshape(NW, n_chunks, chunk)
    dst = edge_index[1].reshape(NW, n_chunks, chunk)

    bpad = jnp.pad(batch, (0, N_PAD - batch.shape[0]), constant_values=64)
    batch_pk = jnp.broadcast_to(
        bpad.reshape(-1, 8, 1), (PK_ROWS, 8, 16)).reshape(-1, 128)
    b1t = jnp.tile(b1, 8).reshape(1, 128)

    degp = _sc_degree(dst)
    x3 = x.reshape(-1, 8, 128)
    p1s_pk, dinv_pk = _tc_proj1(x3, W1, degp.reshape(NC, PK_ROWS, 128))
    agg1 = _sc_edge_pass(p1s_pk.reshape(N_PAD, 16), src, dst, "gcn_edges1_sc")
    h1s_pk = _tc_hidden(agg1.reshape(NC, PK_ROWS, 128), p1s_pk, dinv_pk, b1t)
    agg2 = _sc_edge_pass(h1s_pk.reshape(N_PAD, 16), src, dst, "gcn_edges2_sc")
    return _tc_pool(agg2.reshape(NC, PK_ROWS, 128), h1s_pk, dinv_pk,
                    batch_pk, W2, b2.reshape(1, 2))


# speedup vs baseline: 1.0719x; 1.0480x over previous
import functools

import jax
import jax.numpy as jnp
from jax import lax
from jax.experimental import pallas as pl
from jax.experimental.pallas import tpu as pltpu
from jax.experimental.pallas import tpu_sc as plsc

NC = 2
NS = 16
NW = NC * NS
N_PAD = 10240
ROWS_PER_TILE = N_PAD // NS

_sc_mesh = plsc.VectorSubcoreMesh(
    core_axis_name="c", subcore_axis_name="s", num_cores=NC, num_subcores=NS
)
_sc_params = pltpu.CompilerParams(use_tc_tiling_on_sc=False)


def _wid():
    return lax.axis_index("s") * NC + lax.axis_index("c")


K_FLIGHT = 16


def _deg_body(n_chunks, dst_hbm, ones_hbm, zeros_hbm, out_hbm,
              didx_v, ones_v, deg_sh, sem):
    cid = lax.axis_index("c")
    sid = lax.axis_index("s")
    wid = _wid()
    row0 = sid * ROWS_PER_TILE
    pltpu.sync_copy(zeros_hbm, deg_sh.at[pl.ds(row0, ROWS_PER_TILE)])
    pltpu.sync_copy(ones_hbm, ones_v)
    pltpu.sync_copy(dst_hbm.at[wid], didx_v)
    plsc.subcore_barrier()

    n_sup = n_chunks // K_FLIGHT

    def body(s, carry):
        for b in range(K_FLIGHT):
            j = s * K_FLIGHT + b
            pltpu.async_copy(ones_v, deg_sh.at[didx_v.at[j]], sem, add=True)
        for b in range(K_FLIGHT):
            j = s * K_FLIGHT + b
            pltpu.make_async_copy(ones_v, deg_sh.at[didx_v.at[j]], sem).wait()
        return carry

    lax.fori_loop(0, n_sup, body, 0)
    plsc.subcore_barrier()
    pltpu.sync_copy(deg_sh.at[pl.ds(row0, ROWS_PER_TILE)],
                    out_hbm.at[cid, pl.ds(row0, ROWS_PER_TILE)])


def _edge_body(n_chunks, table_hbm, sidx_hbm, didx_hbm, zeros_hbm, out_hbm,
               sidx_v, didx_v, rows_v, table_sh, acc_sh, gsem, ssem):
    cid = lax.axis_index("c")
    sid = lax.axis_index("s")
    wid = _wid()
    row0 = sid * ROWS_PER_TILE
    pltpu.sync_copy(table_hbm.at[pl.ds(row0, ROWS_PER_TILE)],
                    table_sh.at[pl.ds(row0, ROWS_PER_TILE)])
    pltpu.sync_copy(zeros_hbm, acc_sh.at[pl.ds(row0, ROWS_PER_TILE)])
    pltpu.sync_copy(sidx_hbm.at[wid], sidx_v)
    pltpu.sync_copy(didx_hbm.at[wid], didx_v)
    plsc.subcore_barrier()

    n_sup = n_chunks // K_FLIGHT

    for b in range(K_FLIGHT):
        pltpu.async_copy(table_sh.at[sidx_v.at[b]], rows_v.at[0, b], gsem)

    def body(s, carry):
        bank = lax.rem(s, 2)
        nxt = 1 - bank

        @pl.when(s + 1 < n_sup)
        def _():
            for b in range(K_FLIGHT):
                j = (s + 1) * K_FLIGHT + b
                pltpu.async_copy(table_sh.at[sidx_v.at[j]],
                                 rows_v.at[nxt, b], gsem)

        for b in range(K_FLIGHT):
            j = s * K_FLIGHT + b
            pltpu.make_async_copy(table_sh.at[sidx_v.at[j]],
                                  rows_v.at[bank, b], gsem).wait()
            pltpu.async_copy(rows_v.at[bank, b],
                             acc_sh.at[didx_v.at[j]], ssem, add=True)
        for b in range(K_FLIGHT):
            j = s * K_FLIGHT + b
            pltpu.make_async_copy(rows_v.at[bank, b],
                                  acc_sh.at[didx_v.at[j]], ssem).wait()
        return carry

    lax.fori_loop(0, n_sup, body, 0)
    plsc.subcore_barrier()
    pltpu.sync_copy(acc_sh.at[pl.ds(row0, ROWS_PER_TILE)],
                    out_hbm.at[cid, pl.ds(row0, ROWS_PER_TILE)])


def _sc_degree(dst_idx):
    n_chunks, chunk = dst_idx.shape[1], dst_idx.shape[2]
    k = pl.kernel(
        functools.partial(_deg_body, n_chunks),
        out_type=jax.ShapeDtypeStruct((NC, N_PAD, 16), jnp.float32),
        mesh=_sc_mesh,
        scratch_types=[
            pltpu.VMEM((n_chunks, chunk), jnp.int32),
            pltpu.VMEM((chunk, 16), jnp.float32),
            pltpu.VMEM_SHARED((N_PAD, 16), jnp.float32),
            pltpu.SemaphoreType.DMA,
        ],
        compiler_params=_sc_params,
        name="gcn_degree_sc",
    )
    ones = jnp.ones((chunk, 16), jnp.float32)
    zeros = jnp.zeros((ROWS_PER_TILE, 16), jnp.float32)
    return k(dst_idx, ones, zeros)


def _sc_edge_pass(table, src_idx, dst_idx, name):
    n_chunks, chunk = src_idx.shape[1], src_idx.shape[2]
    k = pl.kernel(
        functools.partial(_edge_body, n_chunks),
        out_type=jax.ShapeDtypeStruct((NC, N_PAD, 16), jnp.float32),
        mesh=_sc_mesh,
        scratch_types=[
            pltpu.VMEM((n_chunks, chunk), jnp.int32),
            pltpu.VMEM((n_chunks, chunk), jnp.int32),
            pltpu.VMEM((2, K_FLIGHT, chunk, 16), jnp.float32),
            pltpu.VMEM_SHARED((N_PAD, 16), jnp.float32),
            pltpu.VMEM_SHARED((N_PAD, 16), jnp.float32),
            pltpu.SemaphoreType.DMA,
            pltpu.SemaphoreType.DMA,
        ],
        compiler_params=_sc_params,
        name=name,
    )
    zeros = jnp.zeros((ROWS_PER_TILE * 16 // 128, 128),
                      jnp.float32).reshape(ROWS_PER_TILE, 16)
    return k(table, src_idx, dst_idx, zeros)


_BLK = 2048
_PBLK = _BLK // 8
_GRID = 5
PK_ROWS = N_PAD // 8
N_REAL_PK = 1250


def _proj1_body(x_ref, w_ref, degp_ref, p1s_ref, dinv_ref):
    dinv = lax.rsqrt(degp_ref[0] + degp_ref[1] + 1.0)
    w = w_ref[...]
    for u in range(8):
        pu = jnp.dot(x_ref[:, u, :], w, preferred_element_type=jnp.float32)
        p1s_ref[:, 16 * u:16 * (u + 1)] = dinv[:, 16 * u:16 * (u + 1)] * pu
    dinv_ref[...] = dinv


def _tc_proj1(x3, W1, degp_pk):
    return pl.pallas_call(
        _proj1_body,
        grid=(_GRID,),
        in_specs=[
            pl.BlockSpec((_PBLK, 8, 128), lambda j: (j, 0, 0)),
            pl.BlockSpec((128, 16), lambda j: (0, 0)),
            pl.BlockSpec((NC, _PBLK, 128), lambda j: (0, j, 0)),
        ],
        out_specs=[
            pl.BlockSpec((_PBLK, 128), lambda j: (j, 0)),
            pl.BlockSpec((_PBLK, 128), lambda j: (j, 0)),
        ],
        out_shape=[
            jax.ShapeDtypeStruct((PK_ROWS, 128), jnp.float32),
            jax.ShapeDtypeStruct((PK_ROWS, 128), jnp.float32),
        ],
    )(x3, W1, degp_pk)


def _hidden_body(aggp_ref, p1s_ref, dinv_ref, b1_ref, h1s_ref):
    dinv = dinv_ref[...]
    s1 = dinv * (aggp_ref[0] + aggp_ref[1] + p1s_ref[...]) + b1_ref[...]
    h1s_ref[...] = dinv * jnp.maximum(s1, 0.0)


def _tc_hidden(aggp_pk, p1s_pk, dinv_pk, b1t):
    return pl.pallas_call(
        _hidden_body,
        grid=(_GRID,),
        in_specs=[
            pl.BlockSpec((NC, _PBLK, 128), lambda j: (0, j, 0)),
            pl.BlockSpec((_PBLK, 128), lambda j: (j, 0)),
            pl.BlockSpec((_PBLK, 128), lambda j: (j, 0)),
            pl.BlockSpec((1, 128), lambda j: (0, 0)),
        ],
        out_specs=pl.BlockSpec((_PBLK, 128), lambda j: (j, 0)),
        out_shape=jax.ShapeDtypeStruct((PK_ROWS, 128), jnp.float32),
    )(aggp_pk, p1s_pk, dinv_pk, b1t)


def _pool_body(aggp_ref, h1s_ref, dinv_ref, batch_ref, w2_ref, b2_ref,
               out_ref, sum_acc, cnt_acc):
    j = pl.program_id(0)

    @pl.when(j == 0)
    def _():
        sum_acc[...] = jnp.zeros_like(sum_acc)
        cnt_acc[...] = jnp.zeros_like(cnt_acc)

    dinv = dinv_ref[...]
    s2 = dinv * (aggp_ref[0] + aggp_ref[1] + h1s_ref[...])
    row = j * _PBLK + lax.broadcasted_iota(jnp.int32, (_PBLK, 1), 0)
    s2 = jnp.where(row < N_REAL_PK, s2, 0.0)
    gi = lax.broadcasted_iota(jnp.int32, (_PBLK, 64), 1)
    ones = jnp.ones((_PBLK, 1), jnp.float32)
    for u in range(8):
        mask = (batch_ref[:, 16 * u:16 * u + 1] == gi).astype(jnp.float32)
        sum_acc[...] += lax.dot_general(
            mask, s2[:, 16 * u:16 * (u + 1)], (((0,), (0,)), ((), ())),
            preferred_element_type=jnp.float32)
        cnt_acc[...] += lax.dot_general(
            mask, ones, (((0,), (0,)), ((), ())),
            preferred_element_type=jnp.float32)

    @pl.when(j == pl.num_programs(0) - 1)
    def _():
        pooled = sum_acc[...] / jnp.maximum(cnt_acc[...], 1.0)
        logits = jnp.dot(pooled, w2_ref[...],
                         preferred_element_type=jnp.float32) + b2_ref[...]
        m = jnp.max(logits, axis=1, keepdims=True)
        lse = m + jnp.log(jnp.sum(jnp.exp(logits - m), axis=1, keepdims=True))
        out_ref[...] = logits - lse


def _tc_pool(aggp_pk, h1s_pk, dinv_pk, batch_pk, W2, b2):
    return pl.pallas_call(
        _pool_body,
        grid=(_GRID,),
        in_specs=[
            pl.BlockSpec((NC, _PBLK, 128), lambda j: (0, j, 0)),
            pl.BlockSpec((_PBLK, 128), lambda j: (j, 0)),
            pl.BlockSpec((_PBLK, 128), lambda j: (j, 0)),
            pl.BlockSpec((_PBLK, 128), lambda j: (j, 0)),
            pl.BlockSpec((16, 2), lambda j: (0, 0)),
            pl.BlockSpec((1, 2), lambda j: (0, 0)),
        ],
        out_specs=pl.BlockSpec((64, 2), lambda j: (0, 0)),
        out_shape=jax.ShapeDtypeStruct((64, 2), jnp.float32),
        scratch_shapes=[
            pltpu.VMEM((64, 16), jnp.float32),
            pltpu.VMEM((64, 1), jnp.float32),
        ],
    )(aggp_pk, h1s_pk, dinv_pk, batch_pk, W2, b2)


def kernel(x, edge_index, batch, W1, b1, W2, b2):
    e = edge_index.shape[1]

    per_tile = e // NW
    chunk = 125
    n_chunks = per_tile // chunk
    src = edge_index[0].reshape(NW, n_chunks, chunk)
    dst = edge_index[1].reshape(NW, n_chunks, chunk)

    bpad = jnp.pad(batch, (0, N_PAD - batch.shape[0]), constant_values=64)
    batch_pk = jnp.broadcast_to(
        bpad.reshape(-1, 8, 1), (PK_ROWS, 8, 16)).reshape(-1, 128)
    b1t = jnp.tile(b1, 8).reshape(1, 128)

    degp = _sc_degree(dst)
    x3 = x.reshape(-1, 8, 128)
    p1s_pk, dinv_pk = _tc_proj1(x3, W1, degp.reshape(NC, PK_ROWS, 128))
    agg1 = _sc_edge_pass(p1s_pk.reshape(N_PAD, 16), src, dst, "gcn_edges1_sc")
    h1s_pk = _tc_hidden(agg1.reshape(NC, PK_ROWS, 128), p1s_pk, dinv_pk, b1t)
    agg2 = _sc_edge_pass(h1s_pk.reshape(N_PAD, 16), src, dst, "gcn_edges2_sc")
    return _tc_pool(agg2.reshape(NC, PK_ROWS, 128), h1s_pk, dinv_pk,
                    batch_pk, W2, b2.reshape(1, 2))


# speedup vs baseline: 1.1037x; 1.0297x over previous
import functools

import jax
import jax.numpy as jnp
from jax import lax
from jax.experimental import pallas as pl
from jax.experimental.pallas import tpu as pltpu
from jax.experimental.pallas import tpu_sc as plsc

NC = 2
NS = 16
NW = NC * NS
N_PAD = 10240
ROWS_PER_TILE = N_PAD // NS

_sc_mesh = plsc.VectorSubcoreMesh(
    core_axis_name="c", subcore_axis_name="s", num_cores=NC, num_subcores=NS
)
_sc_params = pltpu.CompilerParams(use_tc_tiling_on_sc=False)


def _wid():
    return lax.axis_index("s") * NC + lax.axis_index("c")


K_FLIGHT = 8


def _deg_body(n_chunks, dst_hbm, ones_hbm, zeros_hbm, out_hbm,
              didx_v, ones_v, deg_sh, sem):
    cid = lax.axis_index("c")
    sid = lax.axis_index("s")
    wid = _wid()
    row0 = sid * ROWS_PER_TILE
    pltpu.sync_copy(zeros_hbm, deg_sh.at[pl.ds(row0, ROWS_PER_TILE)])
    pltpu.sync_copy(ones_hbm, ones_v)
    pltpu.sync_copy(dst_hbm.at[wid], didx_v)
    plsc.subcore_barrier()

    n_sup = n_chunks // K_FLIGHT

    def body(s, carry):
        for b in range(K_FLIGHT):
            j = s * K_FLIGHT + b
            pltpu.async_copy(ones_v, deg_sh.at[didx_v.at[j]], sem, add=True)
        for b in range(K_FLIGHT):
            j = s * K_FLIGHT + b
            pltpu.make_async_copy(ones_v, deg_sh.at[didx_v.at[j]], sem).wait()
        return carry

    lax.fori_loop(0, n_sup, body, 0)
    plsc.subcore_barrier()
    pltpu.sync_copy(deg_sh.at[pl.ds(row0, ROWS_PER_TILE)],
                    out_hbm.at[cid, pl.ds(row0, ROWS_PER_TILE)])


def _edge_body(n_chunks, table_hbm, sidx_hbm, didx_hbm, zeros_hbm, out_hbm,
               sidx_v, didx_v, rows_v, table_sh, acc_sh, gsem, ssem):
    cid = lax.axis_index("c")
    sid = lax.axis_index("s")
    wid = _wid()
    row0 = sid * ROWS_PER_TILE
    pltpu.sync_copy(table_hbm.at[pl.ds(row0, ROWS_PER_TILE)],
                    table_sh.at[pl.ds(row0, ROWS_PER_TILE)])
    pltpu.sync_copy(zeros_hbm, acc_sh.at[pl.ds(row0, ROWS_PER_TILE)])
    pltpu.sync_copy(sidx_hbm.at[wid], sidx_v)
    pltpu.sync_copy(didx_hbm.at[wid], didx_v)
    plsc.subcore_barrier()

    n_sup = n_chunks // K_FLIGHT

    for b in range(K_FLIGHT):
        pltpu.async_copy(table_sh.at[sidx_v.at[b]], rows_v.at[0, b], gsem)

    def body(s, carry):
        bank = lax.rem(s, 3)
        nxt = lax.rem(s + 1, 3)

        @pl.when(s + 1 < n_sup)
        def _():
            for b in range(K_FLIGHT):
                j = (s + 1) * K_FLIGHT + b
                pltpu.async_copy(table_sh.at[sidx_v.at[j]],
                                 rows_v.at[nxt, b], gsem)

        for b in range(K_FLIGHT):
            j = s * K_FLIGHT + b
            pltpu.make_async_copy(table_sh.at[sidx_v.at[j]],
                                  rows_v.at[bank, b], gsem).wait()
            pltpu.async_copy(rows_v.at[bank, b],
                             acc_sh.at[didx_v.at[j]], ssem, add=True)

        @pl.when(s > 0)
        def _():
            prev = lax.rem(s - 1, 3)
            for b in range(K_FLIGHT):
                j = (s - 1) * K_FLIGHT + b
                pltpu.make_async_copy(rows_v.at[prev, b],
                                      acc_sh.at[didx_v.at[j]], ssem).wait()
        return carry

    lax.fori_loop(0, n_sup, body, 0)
    for b in range(K_FLIGHT):
        j = (n_sup - 1) * K_FLIGHT + b
        pltpu.make_async_copy(rows_v.at[lax.rem(n_sup - 1, 3), b],
                              acc_sh.at[didx_v.at[j]], ssem).wait()
    plsc.subcore_barrier()
    pltpu.sync_copy(acc_sh.at[pl.ds(row0, ROWS_PER_TILE)],
                    out_hbm.at[cid, pl.ds(row0, ROWS_PER_TILE)])


def _sc_degree(dst_idx):
    n_chunks, chunk = dst_idx.shape[1], dst_idx.shape[2]
    k = pl.kernel(
        functools.partial(_deg_body, n_chunks),
        out_type=jax.ShapeDtypeStruct((NC, N_PAD, 16), jnp.float32),
        mesh=_sc_mesh,
        scratch_types=[
            pltpu.VMEM((n_chunks, chunk), jnp.int32),
            pltpu.VMEM((chunk, 16), jnp.float32),
            pltpu.VMEM_SHARED((N_PAD, 16), jnp.float32),
            pltpu.SemaphoreType.DMA,
        ],
        compiler_params=_sc_params,
        name="gcn_degree_sc",
    )
    ones = jnp.ones((chunk, 16), jnp.float32)
    zeros = jnp.zeros((ROWS_PER_TILE, 16), jnp.float32)
    return k(dst_idx, ones, zeros)


def _sc_edge_pass(table, src_idx, dst_idx, name):
    n_chunks, chunk = src_idx.shape[1], src_idx.shape[2]
    k = pl.kernel(
        functools.partial(_edge_body, n_chunks),
        out_type=jax.ShapeDtypeStruct((NC, N_PAD, 16), jnp.float32),
        mesh=_sc_mesh,
        scratch_types=[
            pltpu.VMEM((n_chunks, chunk), jnp.int32),
            pltpu.VMEM((n_chunks, chunk), jnp.int32),
            pltpu.VMEM((3, K_FLIGHT, chunk, 16), jnp.float32),
            pltpu.VMEM_SHARED((N_PAD, 16), jnp.float32),
            pltpu.VMEM_SHARED((N_PAD, 16), jnp.float32),
            pltpu.SemaphoreType.DMA,
            pltpu.SemaphoreType.DMA,
        ],
        compiler_params=_sc_params,
        name=name,
    )
    zeros = jnp.zeros((ROWS_PER_TILE * 16 // 128, 128),
                      jnp.float32).reshape(ROWS_PER_TILE, 16)
    return k(table, src_idx, dst_idx, zeros)


_BLK = 2048
_PBLK = _BLK // 8
_GRID = 5
PK_ROWS = N_PAD // 8
N_REAL_PK = 1250


def _matmul_body(x_ref, w_ref, p1_ref):
    w = w_ref[...]
    for u in range(8):
        p1_ref[:, 16 * u:16 * (u + 1)] = jnp.dot(
            x_ref[:, u, :], w, preferred_element_type=jnp.float32)


def _tc_matmul(x3, W1):
    return pl.pallas_call(
        _matmul_body,
        grid=(_GRID,),
        in_specs=[
            pl.BlockSpec((_PBLK, 8, 128), lambda j: (j, 0, 0)),
            pl.BlockSpec((128, 16), lambda j: (0, 0)),
        ],
        out_specs=pl.BlockSpec((_PBLK, 128), lambda j: (j, 0)),
        out_shape=jax.ShapeDtypeStruct((PK_ROWS, 128), jnp.float32),
    )(x3, W1)


def _scale_body(p1_ref, degp_ref, p1s_ref, dinv_ref):
    dinv = lax.rsqrt(degp_ref[0] + degp_ref[1] + 1.0)
    p1s_ref[...] = dinv * p1_ref[...]
    dinv_ref[...] = dinv


def _tc_scale(p1_pk, degp_pk):
    return pl.pallas_call(
        _scale_body,
        grid=(_GRID,),
        in_specs=[
            pl.BlockSpec((_PBLK, 128), lambda j: (j, 0)),
            pl.BlockSpec((NC, _PBLK, 128), lambda j: (0, j, 0)),
        ],
        out_specs=[
            pl.BlockSpec((_PBLK, 128), lambda j: (j, 0)),
            pl.BlockSpec((_PBLK, 128), lambda j: (j, 0)),
        ],
        out_shape=[
            jax.ShapeDtypeStruct((PK_ROWS, 128), jnp.float32),
            jax.ShapeDtypeStruct((PK_ROWS, 128), jnp.float32),
        ],
    )(p1_pk, degp_pk)


def _hidden_body(aggp_ref, p1s_ref, dinv_ref, b1_ref, h1s_ref):
    dinv = dinv_ref[...]
    s1 = dinv * (aggp_ref[0] + aggp_ref[1] + p1s_ref[...]) + b1_ref[...]
    h1s_ref[...] = dinv * jnp.maximum(s1, 0.0)


def _tc_hidden(aggp_pk, p1s_pk, dinv_pk, b1t):
    return pl.pallas_call(
        _hidden_body,
        grid=(_GRID,),
        in_specs=[
            pl.BlockSpec((NC, _PBLK, 128), lambda j: (0, j, 0)),
            pl.BlockSpec((_PBLK, 128), lambda j: (j, 0)),
            pl.BlockSpec((_PBLK, 128), lambda j: (j, 0)),
            pl.BlockSpec((1, 128), lambda j: (0, 0)),
        ],
        out_specs=pl.BlockSpec((_PBLK, 128), lambda j: (j, 0)),
        out_shape=jax.ShapeDtypeStruct((PK_ROWS, 128), jnp.float32),
    )(aggp_pk, p1s_pk, dinv_pk, b1t)


def _pool_body(aggp_ref, h1s_ref, dinv_ref, batch_ref, w2_ref, b2_ref,
               out_ref, sum_acc, cnt_acc):
    j = pl.program_id(0)

    @pl.when(j == 0)
    def _():
        sum_acc[...] = jnp.zeros_like(sum_acc)
        cnt_acc[...] = jnp.zeros_like(cnt_acc)

    dinv = dinv_ref[...]
    s2 = dinv * (aggp_ref[0] + aggp_ref[1] + h1s_ref[...])
    row = j * _PBLK + lax.broadcasted_iota(jnp.int32, (_PBLK, 1), 0)
    s2 = jnp.where(row < N_REAL_PK, s2, 0.0)
    gi = lax.broadcasted_iota(jnp.int32, (_PBLK, 64), 1)
    ones = jnp.ones((_PBLK, 1), jnp.float32)
    for u in range(8):
        mask = (batch_ref[:, 16 * u:16 * u + 1] == gi).astype(jnp.float32)
        sum_acc[...] += lax.dot_general(
            mask, s2[:, 16 * u:16 * (u + 1)], (((0,), (0,)), ((), ())),
            preferred_element_type=jnp.float32)
        cnt_acc[...] += lax.dot_general(
            mask, ones, (((0,), (0,)), ((), ())),
            preferred_element_type=jnp.float32)

    @pl.when(j == pl.num_programs(0) - 1)
    def _():
        pooled = sum_acc[...] / jnp.maximum(cnt_acc[...], 1.0)
        logits = jnp.dot(pooled, w2_ref[...],
                         preferred_element_type=jnp.float32) + b2_ref[...]
        m = jnp.max(logits, axis=1, keepdims=True)
        lse = m + jnp.log(jnp.sum(jnp.exp(logits - m), axis=1, keepdims=True))
        out_ref[...] = logits - lse


def _tc_pool(aggp_pk, h1s_pk, dinv_pk, batch_pk, W2, b2):
    return pl.pallas_call(
        _pool_body,
        grid=(_GRID,),
        in_specs=[
            pl.BlockSpec((NC, _PBLK, 128), lambda j: (0, j, 0)),
            pl.BlockSpec((_PBLK, 128), lambda j: (j, 0)),
            pl.BlockSpec((_PBLK, 128), lambda j: (j, 0)),
            pl.BlockSpec((_PBLK, 128), lambda j: (j, 0)),
            pl.BlockSpec((16, 2), lambda j: (0, 0)),
            pl.BlockSpec((1, 2), lambda j: (0, 0)),
        ],
        out_specs=pl.BlockSpec((64, 2), lambda j: (0, 0)),
        out_shape=jax.ShapeDtypeStruct((64, 2), jnp.float32),
        scratch_shapes=[
            pltpu.VMEM((64, 16), jnp.float32),
            pltpu.VMEM((64, 1), jnp.float32),
        ],
    )(aggp_pk, h1s_pk, dinv_pk, batch_pk, W2, b2)


def kernel(x, edge_index, batch, W1, b1, W2, b2):
    e = edge_index.shape[1]

    per_tile = e // NW
    chunk = 125
    n_chunks = per_tile // chunk
    src = edge_index[0].reshape(NW, n_chunks, chunk)
    dst = edge_index[1].reshape(NW, n_chunks, chunk)

    bpad = jnp.pad(batch, (0, N_PAD - batch.shape[0]), constant_values=64)
    batch_pk = jnp.broadcast_to(
        bpad.reshape(-1, 8, 1), (PK_ROWS, 8, 16)).reshape(-1, 128)
    b1t = jnp.tile(b1, 8).reshape(1, 128)

    x3 = x.reshape(-1, 8, 128)
    p1_pk = _tc_matmul(x3, W1)
    degp = _sc_degree(dst)
    p1s_pk, dinv_pk = _tc_scale(p1_pk, degp.reshape(NC, PK_ROWS, 128))
    agg1 = _sc_edge_pass(p1s_pk.reshape(N_PAD, 16), src, dst, "gcn_edges1_sc")
    h1s_pk = _tc_hidden(agg1.reshape(NC, PK_ROWS, 128), p1s_pk, dinv_pk, b1t)
    agg2 = _sc_edge_pass(h1s_pk.reshape(N_PAD, 16), src, dst, "gcn_edges2_sc")
    return _tc_pool(agg2.reshape(NC, PK_ROWS, 128), h1s_pk, dinv_pk,
                    batch_pk, W2, b2.reshape(1, 2))


# speedup vs baseline: 1.1115x; 1.0070x over previous
import functools

import jax
import jax.numpy as jnp
from jax import lax
from jax.experimental import pallas as pl
from jax.experimental.pallas import tpu as pltpu
from jax.experimental.pallas import tpu_sc as plsc

NC = 2
NS = 16
NW = NC * NS
N_PAD = 10240
ROWS_PER_TILE = N_PAD // NS

_sc_mesh = plsc.VectorSubcoreMesh(
    core_axis_name="c", subcore_axis_name="s", num_cores=NC, num_subcores=NS
)
_sc_params = pltpu.CompilerParams(use_tc_tiling_on_sc=False)


def _wid():
    return lax.axis_index("s") * NC + lax.axis_index("c")


K_FLIGHT = 8


def _deg_body(n_chunks, dst_hbm, ones_hbm, zeros_hbm, out_hbm,
              didx_v, ones_v, deg_sh, sem):
    cid = lax.axis_index("c")
    sid = lax.axis_index("s")
    wid = _wid()
    row0 = sid * ROWS_PER_TILE
    pltpu.sync_copy(zeros_hbm, deg_sh.at[pl.ds(row0, ROWS_PER_TILE)])
    pltpu.sync_copy(ones_hbm, ones_v)
    pltpu.sync_copy(dst_hbm.at[wid], didx_v)
    plsc.subcore_barrier()

    n_sup = n_chunks // K_FLIGHT

    def body(s, carry):
        for b in range(K_FLIGHT):
            j = s * K_FLIGHT + b
            pltpu.async_copy(ones_v, deg_sh.at[didx_v.at[j]], sem, add=True)
        for b in range(K_FLIGHT):
            j = s * K_FLIGHT + b
            pltpu.make_async_copy(ones_v, deg_sh.at[didx_v.at[j]], sem).wait()
        return carry

    lax.fori_loop(0, n_sup, body, 0)
    plsc.subcore_barrier()
    pltpu.sync_copy(deg_sh.at[pl.ds(row0, ROWS_PER_TILE)],
                    out_hbm.at[cid, pl.ds(row0, ROWS_PER_TILE)])


def _edge_body(n_chunks, table_hbm, sidx_hbm, didx_hbm, zeros_hbm, out_hbm,
               sidx_v, didx_v, rows_v, table_sh, acc_sh, gsem, ssem):
    cid = lax.axis_index("c")
    sid = lax.axis_index("s")
    wid = _wid()
    row0 = sid * ROWS_PER_TILE
    pltpu.sync_copy(table_hbm.at[pl.ds(row0, ROWS_PER_TILE)],
                    table_sh.at[pl.ds(row0, ROWS_PER_TILE)])
    pltpu.sync_copy(zeros_hbm, acc_sh.at[pl.ds(row0, ROWS_PER_TILE)])
    pltpu.sync_copy(sidx_hbm.at[wid], sidx_v)
    pltpu.sync_copy(didx_hbm.at[wid], didx_v)
    plsc.subcore_barrier()

    n_sup = n_chunks // K_FLIGHT

    for b in range(K_FLIGHT):
        pltpu.async_copy(table_sh.at[sidx_v.at[b]], rows_v.at[0, b], gsem)

    def body(s, carry):
        bank = lax.rem(s, 3)
        nxt = lax.rem(s + 1, 3)

        @pl.when(s + 1 < n_sup)
        def _():
            for b in range(K_FLIGHT):
                j = (s + 1) * K_FLIGHT + b
                pltpu.async_copy(table_sh.at[sidx_v.at[j]],
                                 rows_v.at[nxt, b], gsem)

        for b in range(K_FLIGHT):
            j = s * K_FLIGHT + b
            pltpu.make_async_copy(table_sh.at[sidx_v.at[j]],
                                  rows_v.at[bank, b], gsem).wait()
            pltpu.async_copy(rows_v.at[bank, b],
                             acc_sh.at[didx_v.at[j]], ssem, add=True)

        @pl.when(s > 0)
        def _():
            prev = lax.rem(s - 1, 3)
            for b in range(K_FLIGHT):
                j = (s - 1) * K_FLIGHT + b
                pltpu.make_async_copy(rows_v.at[prev, b],
                                      acc_sh.at[didx_v.at[j]], ssem).wait()
        return carry

    lax.fori_loop(0, n_sup, body, 0)
    for b in range(K_FLIGHT):
        j = (n_sup - 1) * K_FLIGHT + b
        pltpu.make_async_copy(rows_v.at[lax.rem(n_sup - 1, 3), b],
                              acc_sh.at[didx_v.at[j]], ssem).wait()
    plsc.subcore_barrier()
    pltpu.sync_copy(acc_sh.at[pl.ds(row0, ROWS_PER_TILE)],
                    out_hbm.at[cid, pl.ds(row0, ROWS_PER_TILE)])


def _sc_degree(dst_idx):
    n_chunks, chunk = dst_idx.shape[1], dst_idx.shape[2]
    k = pl.kernel(
        functools.partial(_deg_body, n_chunks),
        out_type=jax.ShapeDtypeStruct((NC, N_PAD, 16), jnp.float32),
        mesh=_sc_mesh,
        scratch_types=[
            pltpu.VMEM((n_chunks, chunk), jnp.int32),
            pltpu.VMEM((chunk, 16), jnp.float32),
            pltpu.VMEM_SHARED((N_PAD, 16), jnp.float32),
            pltpu.SemaphoreType.DMA,
        ],
        compiler_params=_sc_params,
        name="gcn_degree_sc",
    )
    ones = jnp.ones((chunk, 16), jnp.float32)
    zeros = jnp.zeros((ROWS_PER_TILE, 16), jnp.float32)
    return k(dst_idx, ones, zeros)


def _sc_edge_pass(table, src_idx, dst_idx, name):
    n_chunks, chunk = src_idx.shape[1], src_idx.shape[2]
    k = pl.kernel(
        functools.partial(_edge_body, n_chunks),
        out_type=jax.ShapeDtypeStruct((NC, N_PAD, 16), jnp.float32),
        mesh=_sc_mesh,
        scratch_types=[
            pltpu.VMEM((n_chunks, chunk), jnp.int32),
            pltpu.VMEM((n_chunks, chunk), jnp.int32),
            pltpu.VMEM((3, K_FLIGHT, chunk, 16), jnp.float32),
            pltpu.VMEM_SHARED((N_PAD, 16), jnp.float32),
            pltpu.VMEM_SHARED((N_PAD, 16), jnp.float32),
            pltpu.SemaphoreType.DMA,
            pltpu.SemaphoreType.DMA,
        ],
        compiler_params=_sc_params,
        name=name,
    )
    zeros = jnp.zeros((ROWS_PER_TILE * 16 // 128, 128),
                      jnp.float32).reshape(ROWS_PER_TILE, 16)
    return k(table, src_idx, dst_idx, zeros)


_BLK = 2048
_PBLK = _BLK // 8
_GRID = 5
PK_ROWS = N_PAD // 8
N_REAL_PK = 1250


def _matmul_body(x_ref, w_ref, p1_ref):
    w = w_ref[...]
    for u in range(8):
        p1_ref[:, 16 * u:16 * (u + 1)] = jnp.dot(
            x_ref[:, u, :], w, preferred_element_type=jnp.float32)


def _tc_matmul(x3, W1):
    return pl.pallas_call(
        _matmul_body,
        grid=(_GRID,),
        in_specs=[
            pl.BlockSpec((_PBLK, 8, 128), lambda j: (j, 0, 0)),
            pl.BlockSpec((128, 16), lambda j: (0, 0)),
        ],
        out_specs=pl.BlockSpec((_PBLK, 128), lambda j: (j, 0)),
        out_shape=jax.ShapeDtypeStruct((PK_ROWS, 128), jnp.float32),
    )(x3, W1)


def _scale_body(p1_ref, degp_ref, p1s_ref, dinv_ref):
    dinv = lax.rsqrt(degp_ref[0] + degp_ref[1] + 1.0)
    p1s_ref[...] = dinv * p1_ref[...]
    dinv_ref[...] = dinv


def _tc_scale(p1_pk, degp_pk):
    return pl.pallas_call(
        _scale_body,
        grid=(_GRID,),
        in_specs=[
            pl.BlockSpec((_PBLK, 128), lambda j: (j, 0)),
            pl.BlockSpec((NC, _PBLK, 128), lambda j: (0, j, 0)),
        ],
        out_specs=[
            pl.BlockSpec((_PBLK, 128), lambda j: (j, 0)),
            pl.BlockSpec((_PBLK, 128), lambda j: (j, 0)),
        ],
        out_shape=[
            jax.ShapeDtypeStruct((PK_ROWS, 128), jnp.float32),
            jax.ShapeDtypeStruct((PK_ROWS, 128), jnp.float32),
        ],
    )(p1_pk, degp_pk)


def _hidden_body(aggp_ref, p1s_ref, dinv_ref, b1_ref, h1s_ref):
    dinv = dinv_ref[...]
    s1 = dinv * (aggp_ref[0] + aggp_ref[1] + p1s_ref[...]) + b1_ref[...]
    h1s_ref[...] = dinv * jnp.maximum(s1, 0.0)


def _tc_hidden(aggp_pk, p1s_pk, dinv_pk, b1t):
    return pl.pallas_call(
        _hidden_body,
        grid=(_GRID,),
        in_specs=[
            pl.BlockSpec((NC, _PBLK, 128), lambda j: (0, j, 0)),
            pl.BlockSpec((_PBLK, 128), lambda j: (j, 0)),
            pl.BlockSpec((_PBLK, 128), lambda j: (j, 0)),
            pl.BlockSpec((1, 128), lambda j: (0, 0)),
        ],
        out_specs=pl.BlockSpec((_PBLK, 128), lambda j: (j, 0)),
        out_shape=jax.ShapeDtypeStruct((PK_ROWS, 128), jnp.float32),
    )(aggp_pk, p1s_pk, dinv_pk, b1t)


def _pool_body(aggp_ref, h1s_ref, dinv_ref, batch_ref, w2_ref, b2_ref,
               out_ref):
    dinv = dinv_ref[...]
    s2 = dinv * (aggp_ref[0] + aggp_ref[1] + h1s_ref[...])
    row = lax.broadcasted_iota(jnp.int32, (PK_ROWS, 1), 0)
    s2 = jnp.where(row < N_REAL_PK, s2, 0.0)
    gi = lax.broadcasted_iota(jnp.int32, (PK_ROWS, 64), 1)
    ones = jnp.ones((PK_ROWS, 1), jnp.float32)
    sums = jnp.zeros((64, 16), jnp.float32)
    cnts = jnp.zeros((64, 1), jnp.float32)
    for u in range(8):
        mask = (batch_ref[:, 16 * u:16 * u + 1] == gi).astype(jnp.float32)
        sums += lax.dot_general(
            mask, s2[:, 16 * u:16 * (u + 1)], (((0,), (0,)), ((), ())),
            preferred_element_type=jnp.float32)
        cnts += lax.dot_general(
            mask, ones, (((0,), (0,)), ((), ())),
            preferred_element_type=jnp.float32)
    pooled = sums / jnp.maximum(cnts, 1.0)
    logits = jnp.dot(pooled, w2_ref[...],
                     preferred_element_type=jnp.float32) + b2_ref[...]
    m = jnp.max(logits, axis=1, keepdims=True)
    lse = m + jnp.log(jnp.sum(jnp.exp(logits - m), axis=1, keepdims=True))
    out_ref[...] = logits - lse


def _tc_pool(aggp_pk, h1s_pk, dinv_pk, batch_pk, W2, b2):
    return pl.pallas_call(
        _pool_body,
        in_specs=[
            pl.BlockSpec((NC, PK_ROWS, 128), lambda: (0, 0, 0)),
            pl.BlockSpec((PK_ROWS, 128), lambda: (0, 0)),
            pl.BlockSpec((PK_ROWS, 128), lambda: (0, 0)),
            pl.BlockSpec((PK_ROWS, 128), lambda: (0, 0)),
            pl.BlockSpec((16, 2), lambda: (0, 0)),
            pl.BlockSpec((1, 2), lambda: (0, 0)),
        ],
        out_specs=pl.BlockSpec((64, 2), lambda: (0, 0)),
        out_shape=jax.ShapeDtypeStruct((64, 2), jnp.float32),
    )(aggp_pk, h1s_pk, dinv_pk, batch_pk, W2, b2)


def kernel(x, edge_index, batch, W1, b1, W2, b2):
    e = edge_index.shape[1]

    per_tile = e // NW
    chunk = 125
    n_chunks = per_tile // chunk
    src = edge_index[0].reshape(NW, n_chunks, chunk)
    dst = edge_index[1].reshape(NW, n_chunks, chunk)

    bpad = jnp.pad(batch, (0, N_PAD - batch.shape[0]), constant_values=64)
    batch_pk = jnp.broadcast_to(
        bpad.reshape(-1, 8, 1), (PK_ROWS, 8, 16)).reshape(-1, 128)
    b1t = jnp.tile(b1, 8).reshape(1, 128)

    x3 = x.reshape(-1, 8, 128)
    p1_pk = _tc_matmul(x3, W1)
    degp = _sc_degree(dst)
    p1s_pk, dinv_pk = _tc_scale(p1_pk, degp.reshape(NC, PK_ROWS, 128))
    agg1 = _sc_edge_pass(p1s_pk.reshape(N_PAD, 16), src, dst, "gcn_edges1_sc")
    h1s_pk = _tc_hidden(agg1.reshape(NC, PK_ROWS, 128), p1s_pk, dinv_pk, b1t)
    agg2 = _sc_edge_pass(h1s_pk.reshape(N_PAD, 16), src, dst, "gcn_edges2_sc")
    return _tc_pool(agg2.reshape(NC, PK_ROWS, 128), h1s_pk, dinv_pk,
                    batch_pk, W2, b2.reshape(1, 2))
